# baseline (speedup 1.0000x reference)
import jax
import jax.numpy as jnp
from jax.experimental import pallas as pl
from jax.experimental.pallas import tpu as pltpu

_N = 8192
_R = 64
_L = 128
_U = 128


def _vq_kernel(w_ref, x_ref, val_ref, idx_ref):
    xv = x_ref[...]

    def body(t, carry):
        bd, bj, bv = carry
        wrow = w_ref[t]
        for u in range(_U):
            j = t * _U + u
            c = jnp.broadcast_to(wrow[u], (_R, _L))
            d = xv - c
            d = d * d
            m = d < bd
            bd = jnp.where(m, d, bd)
            bj = jnp.where(m, j, bj)
            bv = jnp.where(m, c, bv)
        return bd, bj, bv

    bd0 = jnp.full((_R, _L), jnp.inf, jnp.float32)
    bj0 = jnp.zeros((_R, _L), jnp.int32)
    bv0 = jnp.zeros((_R, _L), jnp.float32)
    _, bj, bv = jax.lax.fori_loop(0, _N // _U, body, (bd0, bj0, bv0))

    idx_ref[...] = bj
    val_ref[...] = bv


def kernel(x, weight):
    shape = x.shape
    xf = x.reshape(_R, _L)
    wf = weight.reshape(_N // _U, _U)
    val, idx = pl.pallas_call(
        _vq_kernel,
        in_specs=[
            pl.BlockSpec(memory_space=pltpu.MemorySpace.VMEM),
            pl.BlockSpec(memory_space=pltpu.MemorySpace.VMEM),
        ],
        out_specs=[
            pl.BlockSpec(memory_space=pltpu.MemorySpace.VMEM),
            pl.BlockSpec(memory_space=pltpu.MemorySpace.VMEM),
        ],
        out_shape=[
            jax.ShapeDtypeStruct((_R, _L), jnp.float32),
            jax.ShapeDtypeStruct((_R, _L), jnp.int32),
        ],
    )(wf, xf)
    return val.reshape(shape), idx.reshape(shape)


# speedup vs baseline: 1.1534x; 1.1534x over previous
import jax
import jax.numpy as jnp
from jax.experimental import pallas as pl
from jax.experimental.pallas import tpu as pltpu

_N = 8192
_R = 64
_L = 128
_U = 64


def _vq_kernel(w_ref, x_ref, val_ref, idx_ref):
    xv = jnp.concatenate(
        [
            jnp.concatenate(
                [x_ref[b, 0, 8 * g : 8 * g + 8, :] for g in range(4)], axis=1
            )
            for b in range(8)
        ],
        axis=0,
    )

    def body(t, carry):
        bd, bj, bv = carry
        for u in range(_U):
            j = t * _U + u
            c = w_ref[0, j]
            d = xv - c
            d = d * d
            m = d < bd
            bd = jnp.where(m, d, bd)
            bj = jnp.where(m, j, bj)
            bv = jnp.where(m, c, bv)
        return bd, bj, bv

    bd0 = jnp.full((_R, _L), jnp.inf, jnp.float32)
    bj0 = jnp.zeros((_R, _L), jnp.int32)
    bv0 = jnp.zeros((_R, _L), jnp.float32)
    _, bj, bv = jax.lax.fori_loop(0, _N // _U, body, (bd0, bj0, bv0))

    for b in range(8):
        idx_ref[b, 0] = jnp.concatenate(
            [bj[8 * b : 8 * b + 8, 32 * g : 32 * g + 32] for g in range(4)],
            axis=0,
        )
        val_ref[b, 0] = jnp.concatenate(
            [bv[8 * b : 8 * b + 8, 32 * g : 32 * g + 32] for g in range(4)],
            axis=0,
        )


def kernel(x, weight):
    val, idx = pl.pallas_call(
        _vq_kernel,
        in_specs=[
            pl.BlockSpec(memory_space=pltpu.MemorySpace.SMEM),
            pl.BlockSpec(memory_space=pltpu.MemorySpace.VMEM),
        ],
        out_specs=[
            pl.BlockSpec(memory_space=pltpu.MemorySpace.VMEM),
            pl.BlockSpec(memory_space=pltpu.MemorySpace.VMEM),
        ],
        out_shape=[
            jax.ShapeDtypeStruct(x.shape, jnp.float32),
            jax.ShapeDtypeStruct(x.shape, jnp.int32),
        ],
    )(weight, x)
    return val, idx


# speedup vs baseline: 1.1641x; 1.0093x over previous
import jax
import jax.numpy as jnp
from jax.experimental import pallas as pl
from jax.experimental.pallas import tpu as pltpu

_N = 8192
_R = 64
_L = 128
_U = 128


def _vq_kernel(w_ref, x_ref, val_ref, idx_ref):
    xv = jnp.concatenate(
        [
            jnp.concatenate(
                [x_ref[b, 0, 8 * g : 8 * g + 8, :] for g in range(4)], axis=1
            )
            for b in range(8)
        ],
        axis=0,
    )

    def body(t, carry):
        bd, bj, bv = carry
        for u in range(_U):
            j = t * _U + u
            c = w_ref[0, j]
            d = xv - c
            d = d * d
            m = d < bd
            bd = jnp.where(m, d, bd)
            bj = jnp.where(m, j, bj)
            bv = jnp.where(m, c, bv)
        return bd, bj, bv

    bd0 = jnp.full((_R, _L), jnp.inf, jnp.float32)
    bj0 = jnp.zeros((_R, _L), jnp.int32)
    bv0 = jnp.zeros((_R, _L), jnp.float32)
    _, bj, bv = jax.lax.fori_loop(0, _N // _U, body, (bd0, bj0, bv0))

    for b in range(8):
        idx_ref[b, 0] = jnp.concatenate(
            [bj[8 * b : 8 * b + 8, 32 * g : 32 * g + 32] for g in range(4)],
            axis=0,
        )
        val_ref[b, 0] = jnp.concatenate(
            [bv[8 * b : 8 * b + 8, 32 * g : 32 * g + 32] for g in range(4)],
            axis=0,
        )


def kernel(x, weight):
    val, idx = pl.pallas_call(
        _vq_kernel,
        in_specs=[
            pl.BlockSpec(memory_space=pltpu.MemorySpace.SMEM),
            pl.BlockSpec(memory_space=pltpu.MemorySpace.VMEM),
        ],
        out_specs=[
            pl.BlockSpec(memory_space=pltpu.MemorySpace.VMEM),
            pl.BlockSpec(memory_space=pltpu.MemorySpace.VMEM),
        ],
        out_shape=[
            jax.ShapeDtypeStruct(x.shape, jnp.float32),
            jax.ShapeDtypeStruct(x.shape, jnp.int32),
        ],
    )(weight, x)
    return val, idx
